# baseline (speedup 1.0000x reference)
import functools

import jax
import jax.numpy as jnp
from jax import lax
from jax.experimental import pallas as pl
from jax.experimental.pallas import tpu as pltpu
from jax.experimental.pallas import tpu_sc as plsc

_N = 8192
_IN = 128
_L = 256
_K = 16
_B = 8
_EPS = 1e-5

_BIG = 3.0e38
_IMAX = 2**31 - 1

_RB = 512
_CT = 1024
_NRB = _N // _RB
_NCT = _N // _CT


def _topk_body(jlo_ref, jhi_ref, xi_ref, xt_ref, bi_ref, bj_ref, out_ref,
               topv, topi):
    i = pl.program_id(0)
    j = pl.program_id(1)

    @pl.when(j == 0)
    def _init():
        topv[...] = jnp.full((_RB, _K), _BIG, jnp.float32)
        topi[...] = jnp.full((_RB, _K), _BIG, jnp.float32)

    active = jnp.logical_and(j >= jlo_ref[i], j <= jhi_ref[i])

    @pl.when(active)
    def _work():
        xi = xi_ref[...]
        xt = xt_ref[...]
        d2 = lax.dot_general(xi, xt, (((1,), (0,)), ((), ())),
                             preferred_element_type=jnp.float32)
        sqi = jnp.sum(xi * xi, axis=1, keepdims=True)
        sqj = jnp.sum(xt * xt, axis=0, keepdims=True)
        d = sqi + sqj - 2.0 * d2
        mask = bi_ref[...] != bj_ref[...]
        d = jnp.where(mask, _BIG, d)
        colidx = ((j * _CT).astype(jnp.float32)
                  + lax.broadcasted_iota(jnp.int32, (_RB, _CT), 1
                                         ).astype(jnp.float32))

        c = jnp.concatenate([topv[...], d], axis=1)
        ci = jnp.concatenate([topi[...], colidx], axis=1)
        vs = []
        gs = []
        for _ in range(_K):
            m = jnp.min(c, axis=1, keepdims=True)
            sel = c == m
            gi = jnp.min(jnp.where(sel, ci, _BIG), axis=1, keepdims=True)
            vs.append(m)
            gs.append(gi)
            c = jnp.where(ci == gi, _BIG, c)
        topv[...] = jnp.concatenate(vs, axis=1)
        topi[...] = jnp.concatenate(gs, axis=1)

    @pl.when(j == _NCT - 1)
    def _fin():
        out_ref[...] = topi[...].astype(jnp.int32)


def _knn(x, xt, b_col, b_row, jlo, jhi):
    grid_spec = pltpu.PrefetchScalarGridSpec(
        num_scalar_prefetch=2,
        grid=(_NRB, _NCT),
        in_specs=[
            pl.BlockSpec((_RB, _IN), lambda i, j, jlo, jhi: (i, 0)),
            pl.BlockSpec((_IN, _CT),
                         lambda i, j, jlo, jhi: (0, jnp.clip(j, jlo[i], jhi[i]))),
            pl.BlockSpec((_RB, 1), lambda i, j, jlo, jhi: (i, 0)),
            pl.BlockSpec((1, _CT),
                         lambda i, j, jlo, jhi: (0, jnp.clip(j, jlo[i], jhi[i]))),
        ],
        out_specs=pl.BlockSpec((_RB, _K), lambda i, j, jlo, jhi: (i, 0)),
        scratch_shapes=[pltpu.VMEM((_RB, _K), jnp.float32),
                        pltpu.VMEM((_RB, _K), jnp.float32)],
    )
    return pl.pallas_call(
        _topk_body,
        grid_spec=grid_spec,
        out_shape=jax.ShapeDtypeStruct((_N, _K), jnp.int32),
    )(jlo, jhi, x, xt, b_col, b_row)


_MB = 512


def _mm_body(x_ref, w_ref, y_ref):
    y_ref[...] = lax.dot_general(x_ref[...], w_ref[...], (((1,), (0,)), ((), ())),
                                 preferred_element_type=jnp.float32)


def _ab_matmul(x, wc):
    return pl.pallas_call(
        _mm_body,
        grid=(_N // _MB,),
        in_specs=[pl.BlockSpec((_MB, _IN), lambda i: (i, 0)),
                  pl.BlockSpec((_IN, 2 * _L), lambda i: (0, 0))],
        out_specs=pl.BlockSpec((_MB, 2 * _L), lambda i: (i, 0)),
        out_shape=jax.ShapeDtypeStruct((_N, 2 * _L), jnp.float32),
    )(x, wc)


_NW = 32
_EPW = _N * _K // _NW
_PPW = _N // _NW
_CH = 128
_NP = _EPW // (2 * _CH)


def _gather_body(table_hbm, idx_hbm, out_hbm, idx_v, rows_a, rows_b,
                 sem_a, sem_b):
    wid = lax.axis_index("s") * 2 + lax.axis_index("c")
    i0 = wid * _PPW
    for k in range(_K):
        pltpu.sync_copy(idx_hbm.at[pl.ds(k * _N + i0, _PPW)],
                        idx_v.at[pl.ds(k * _PPW, _PPW)])

    def dst(c):
        k = c // 2
        half = c - 2 * k
        return k * _N + i0 + half * _CH

    pltpu.async_copy(table_hbm.at[idx_v.at[pl.ds(0, _CH)]], rows_a, sem_a)

    def body(p, carry):
        c0 = 2 * p
        c1 = c0 + 1
        pltpu.make_async_copy(table_hbm.at[idx_v.at[pl.ds(0, _CH)]],
                              rows_a, sem_a).wait()
        pltpu.async_copy(table_hbm.at[idx_v.at[pl.ds(c1 * _CH, _CH)]],
                         rows_b, sem_b)
        pltpu.sync_copy(rows_a, out_hbm.at[pl.ds(dst(c0), _CH)])
        pltpu.make_async_copy(table_hbm.at[idx_v.at[pl.ds(0, _CH)]],
                              rows_b, sem_b).wait()

        @pl.when(p + 1 < _NP)
        def _next():
            pltpu.async_copy(table_hbm.at[idx_v.at[pl.ds((c0 + 2) * _CH, _CH)]],
                             rows_a, sem_a)

        pltpu.sync_copy(rows_b, out_hbm.at[pl.ds(dst(c1), _CH)])
        return carry

    lax.fori_loop(0, _NP, body, 0)


def _sc_gather(table, idx_flat):
    return pl.kernel(
        _gather_body,
        out_type=jax.ShapeDtypeStruct((_N * _K, _L), jnp.float32),
        mesh=plsc.VectorSubcoreMesh(core_axis_name="c", subcore_axis_name="s"),
        scratch_types=[pltpu.VMEM((_EPW,), jnp.int32),
                       pltpu.VMEM((_CH, _L), jnp.float32),
                       pltpu.VMEM((_CH, _L), jnp.float32),
                       pltpu.SemaphoreType.DMA,
                       pltpu.SemaphoreType.DMA],
    )(table, idx_flat)


_PB4 = 128
_N4 = _N // _PB4


def _bn_out_body(g_ref, a_ref, b_ref, gam_ref, bet_ref, o_ref,
                 acc1, acc2, st):
    i = pl.program_id(0)

    @pl.when(i == 0)
    def _init():
        acc1[...] = jnp.zeros((_PB4, _L), jnp.float32)
        acc2[...] = jnp.zeros((_PB4, _L), jnp.float32)

    a = a_ref[...] + b_ref[...]

    @pl.when(i < _N4)
    def _stats():
        gsum = jnp.zeros((_PB4, _L), jnp.float32)
        h2 = jnp.zeros((_PB4, _L), jnp.float32)
        for k in range(_K):
            gk = g_ref[k]
            gsum = gsum + gk
            h = a + gk
            h2 = h2 + h * h
        acc1[...] += gsum + float(_K) * a
        acc2[...] += h2

    @pl.when(i == _N4)
    def _mkst():
        nk = float(_N * _K)
        mu = jnp.sum(acc1[...], axis=0, keepdims=True) * (1.0 / nk)
        var = jnp.sum(acc2[...], axis=0, keepdims=True) * (1.0 / nk) - mu * mu
        s = gam_ref[...] * lax.rsqrt(var + _EPS)
        st[0:1, :] = s
        st[1:2, :] = bet_ref[...] - mu * s

    @pl.when(i >= _N4)
    def _emit():
        s = st[0:1, :]
        t = st[1:2, :]
        u = a * s + t
        acc = jnp.zeros((_PB4, _L), jnp.float32)
        for k in range(_K):
            acc = acc + jnp.maximum(g_ref[k] * s + u, 0.0)
        o_ref[...] = acc * (1.0 / _K)


def _bn_out(g3, a, b2, gam2, bet2):
    return pl.pallas_call(
        _bn_out_body,
        grid=(2 * _N4,),
        in_specs=[pl.BlockSpec((_K, _PB4, _L), lambda i: (0, i % _N4, 0)),
                  pl.BlockSpec((_PB4, _L), lambda i: (i % _N4, 0)),
                  pl.BlockSpec((1, _L), lambda i: (0, 0)),
                  pl.BlockSpec((1, _L), lambda i: (0, 0)),
                  pl.BlockSpec((1, _L), lambda i: (0, 0))],
        out_specs=pl.BlockSpec((_PB4, _L),
                               lambda i: (jnp.maximum(i - _N4, 0), 0)),
        out_shape=jax.ShapeDtypeStruct((_N, _L), jnp.float32),
        scratch_shapes=[pltpu.VMEM((_PB4, _L), jnp.float32),
                        pltpu.VMEM((_PB4, _L), jnp.float32),
                        pltpu.VMEM((8, _L), jnp.float32)],
    )(g3, a, b2, gam2, bet2)


def kernel(x, batch, W, b, gamma, beta):
    batch = batch.astype(jnp.int32)
    xt = x.T
    b_col = batch.reshape(_N, 1)
    b_row = batch.reshape(1, _N)

    bids = jnp.arange(_B, dtype=jnp.int32)
    starts = jnp.searchsorted(batch, bids, side="left").astype(jnp.int32)
    ends = jnp.searchsorted(batch, bids, side="right").astype(jnp.int32)
    bfirst = batch[::_RB]
    blast = batch[_RB - 1::_RB]
    jlo = (starts[bfirst] // _CT).astype(jnp.int32)
    jhi = ((ends[blast] - 1) // _CT).astype(jnp.int32)

    idx = _knn(x, xt, b_col, b_row, jlo, jhi)

    w2 = W[:, _IN:]
    wc = jnp.concatenate([(W[:, :_IN] - w2).T, w2.T], axis=1)
    y = _ab_matmul(x, wc)
    a = y[:, :_L]
    bm = y[:, _L:]

    g = _sc_gather(bm, idx.T.reshape(-1))
    g3 = g.reshape(_K, _N, _L)

    return _bn_out(g3, a, b.reshape(1, _L), gamma.reshape(1, _L),
                   beta.reshape(1, _L))


# speedup vs baseline: 1.1406x; 1.1406x over previous
import functools

import jax
import jax.numpy as jnp
from jax import lax
from jax.experimental import pallas as pl
from jax.experimental.pallas import tpu as pltpu
from jax.experimental.pallas import tpu_sc as plsc

_N = 8192
_IN = 128
_L = 256
_K = 16
_B = 8
_EPS = 1e-5

_BIG = 3.0e38
_IMAX = 2**31 - 1

_RB = 512
_CT = 512
_NRB = _N // _RB
_NCT = _N // _CT


def _topk_body(jlo_ref, jhi_ref, xi_ref, xt_ref, bi_ref, bj_ref, out_ref,
               topv, topi):
    i = pl.program_id(0)
    j = pl.program_id(1)

    @pl.when(j == 0)
    def _init():
        topv[...] = jnp.full((_RB, _K), _BIG, jnp.float32)
        topi[...] = jnp.full((_RB, _K), _BIG, jnp.float32)

    active = jnp.logical_and(j >= jlo_ref[i], j <= jhi_ref[i])

    @pl.when(active)
    def _work():
        xi = xi_ref[...]
        xt = xt_ref[...]
        d2 = lax.dot_general(xi, xt, (((1,), (0,)), ((), ())),
                             preferred_element_type=jnp.float32)
        sqi = jnp.sum(xi * xi, axis=1, keepdims=True)
        sqj = jnp.sum(xt * xt, axis=0, keepdims=True)
        d = sqi + sqj - 2.0 * d2
        mask = bi_ref[...] != bj_ref[...]
        d = jnp.where(mask, _BIG, d)
        colidx = ((j * _CT).astype(jnp.float32)
                  + lax.broadcasted_iota(jnp.int32, (_RB, _CT), 1
                                         ).astype(jnp.float32))

        c = jnp.concatenate([topv[...], d], axis=1)
        ci = jnp.concatenate([topi[...], colidx], axis=1)
        vs = []
        gs = []
        for _ in range(_K):
            m = jnp.min(c, axis=1, keepdims=True)
            sel = c == m
            gi = jnp.min(jnp.where(sel, ci, _BIG), axis=1, keepdims=True)
            vs.append(m)
            gs.append(gi)
            c = jnp.where(ci == gi, _BIG, c)
        topv[...] = jnp.concatenate(vs, axis=1)
        topi[...] = jnp.concatenate(gs, axis=1)

    @pl.when(j == _NCT - 1)
    def _fin():
        out_ref[...] = topi[...].astype(jnp.int32)


def _knn(x, xt, b_col, b_row, jlo, jhi):
    grid_spec = pltpu.PrefetchScalarGridSpec(
        num_scalar_prefetch=2,
        grid=(_NRB, _NCT),
        in_specs=[
            pl.BlockSpec((_RB, _IN), lambda i, j, jlo, jhi: (i, 0)),
            pl.BlockSpec((_IN, _CT),
                         lambda i, j, jlo, jhi: (0, jnp.clip(j, jlo[i], jhi[i]))),
            pl.BlockSpec((_RB, 1), lambda i, j, jlo, jhi: (i, 0)),
            pl.BlockSpec((1, _CT),
                         lambda i, j, jlo, jhi: (0, jnp.clip(j, jlo[i], jhi[i]))),
        ],
        out_specs=pl.BlockSpec((_RB, _K), lambda i, j, jlo, jhi: (i, 0)),
        scratch_shapes=[pltpu.VMEM((_RB, _K), jnp.float32),
                        pltpu.VMEM((_RB, _K), jnp.float32)],
    )
    return pl.pallas_call(
        _topk_body,
        grid_spec=grid_spec,
        out_shape=jax.ShapeDtypeStruct((_N, _K), jnp.int32),
    )(jlo, jhi, x, xt, b_col, b_row)


_MB = 512


_LH = _L // 2


def _f32_to_bf16_bits(f):
    bits = lax.bitcast_convert_type(f, jnp.int32)
    rnd = (bits + 0x7FFF + ((bits >> 16) & 1)) >> 16
    return rnd & 0xFFFF


def _mm_body(x_ref, w_ref, a_ref, bm_ref):
    y = lax.dot_general(x_ref[...], w_ref[...], (((1,), (0,)), ((), ())),
                        preferred_element_type=jnp.float32)
    a_ref[...] = y[:, :_L]
    b1 = _f32_to_bf16_bits(y[:, _L:_L + _LH])
    b2 = _f32_to_bf16_bits(y[:, _L + _LH:])
    bm_ref[...] = b1 | (b2 << 16)


def _ab_matmul(x, wc):
    return pl.pallas_call(
        _mm_body,
        grid=(_N // _MB,),
        in_specs=[pl.BlockSpec((_MB, _IN), lambda i: (i, 0)),
                  pl.BlockSpec((_IN, 2 * _L), lambda i: (0, 0))],
        out_specs=[pl.BlockSpec((_MB, _L), lambda i: (i, 0)),
                   pl.BlockSpec((_MB, _LH), lambda i: (i, 0))],
        out_shape=[jax.ShapeDtypeStruct((_N, _L), jnp.float32),
                   jax.ShapeDtypeStruct((_N, _LH), jnp.int32)],
    )(x, wc)


_NW = 32
_EPW = _N * _K // _NW
_PPW = _N // _NW
_CH = 128
_NP = _EPW // (2 * _CH)


def _gather_body(table_hbm, idx_hbm, out_hbm, idx_v, rows_a, rows_b,
                 sem_a, sem_b):
    wid = lax.axis_index("s") * 2 + lax.axis_index("c")
    i0 = wid * _PPW
    for k in range(_K):
        pltpu.sync_copy(idx_hbm.at[pl.ds(k * _N + i0, _PPW)],
                        idx_v.at[pl.ds(k * _PPW, _PPW)])

    def dst(c):
        k = c // 2
        half = c - 2 * k
        return k * _N + i0 + half * _CH

    pltpu.async_copy(table_hbm.at[idx_v.at[pl.ds(0, _CH)]], rows_a, sem_a)

    def body(p, carry):
        c0 = 2 * p
        c1 = c0 + 1
        pltpu.make_async_copy(table_hbm.at[idx_v.at[pl.ds(0, _CH)]],
                              rows_a, sem_a).wait()
        pltpu.async_copy(table_hbm.at[idx_v.at[pl.ds(c1 * _CH, _CH)]],
                         rows_b, sem_b)
        pltpu.sync_copy(rows_a, out_hbm.at[pl.ds(dst(c0), _CH)])
        pltpu.make_async_copy(table_hbm.at[idx_v.at[pl.ds(0, _CH)]],
                              rows_b, sem_b).wait()

        @pl.when(p + 1 < _NP)
        def _next():
            pltpu.async_copy(table_hbm.at[idx_v.at[pl.ds((c0 + 2) * _CH, _CH)]],
                             rows_a, sem_a)

        pltpu.sync_copy(rows_b, out_hbm.at[pl.ds(dst(c1), _CH)])
        return carry

    lax.fori_loop(0, _NP, body, 0)


def _sc_gather(table, idx_flat):
    return pl.kernel(
        _gather_body,
        out_type=jax.ShapeDtypeStruct((_N * _K, _LH), jnp.int32),
        mesh=plsc.VectorSubcoreMesh(core_axis_name="c", subcore_axis_name="s"),
        scratch_types=[pltpu.VMEM((_EPW,), jnp.int32),
                       pltpu.VMEM((_CH, _LH), jnp.int32),
                       pltpu.VMEM((_CH, _LH), jnp.int32),
                       pltpu.SemaphoreType.DMA,
                       pltpu.SemaphoreType.DMA],
    )(table, idx_flat)


_PB4 = 128
_N4 = _N // _PB4


def _unpack_bf16x2(gk):
    lo = lax.bitcast_convert_type(gk << 16, jnp.float32)
    hi = lax.bitcast_convert_type(gk & jnp.int32(-65536), jnp.float32)
    return jnp.concatenate([lo, hi], axis=1)


def _bn_out_body(g_ref, a_ref, b_ref, gam_ref, bet_ref, o_ref,
                 acc1, acc2, st):
    i = pl.program_id(0)

    @pl.when(i == 0)
    def _init():
        acc1[...] = jnp.zeros((_PB4, _L), jnp.float32)
        acc2[...] = jnp.zeros((_PB4, _L), jnp.float32)

    a = a_ref[...] + b_ref[...]

    @pl.when(i < _N4)
    def _stats():
        gsum = jnp.zeros((_PB4, _L), jnp.float32)
        h2 = jnp.zeros((_PB4, _L), jnp.float32)
        for k in range(_K):
            gk = _unpack_bf16x2(g_ref[k])
            gsum = gsum + gk
            h = a + gk
            h2 = h2 + h * h
        acc1[...] += gsum + float(_K) * a
        acc2[...] += h2

    @pl.when(i == _N4)
    def _mkst():
        nk = float(_N * _K)
        mu = jnp.sum(acc1[...], axis=0, keepdims=True) * (1.0 / nk)
        var = jnp.sum(acc2[...], axis=0, keepdims=True) * (1.0 / nk) - mu * mu
        s = gam_ref[...] * lax.rsqrt(var + _EPS)
        st[0:1, :] = s
        st[1:2, :] = bet_ref[...] - mu * s

    @pl.when(i >= _N4)
    def _emit():
        s = st[0:1, :]
        t = st[1:2, :]
        u = a * s + t
        acc = jnp.zeros((_PB4, _L), jnp.float32)
        for k in range(_K):
            acc = acc + jnp.maximum(_unpack_bf16x2(g_ref[k]) * s + u, 0.0)
        o_ref[...] = acc * (1.0 / _K)


def _bn_out(g3, a, b2, gam2, bet2):
    return pl.pallas_call(
        _bn_out_body,
        grid=(2 * _N4,),
        in_specs=[pl.BlockSpec((_K, _PB4, _LH), lambda i: (0, i % _N4, 0)),
                  pl.BlockSpec((_PB4, _L), lambda i: (i % _N4, 0)),
                  pl.BlockSpec((1, _L), lambda i: (0, 0)),
                  pl.BlockSpec((1, _L), lambda i: (0, 0)),
                  pl.BlockSpec((1, _L), lambda i: (0, 0))],
        out_specs=pl.BlockSpec((_PB4, _L),
                               lambda i: (jnp.maximum(i - _N4, 0), 0)),
        out_shape=jax.ShapeDtypeStruct((_N, _L), jnp.float32),
        scratch_shapes=[pltpu.VMEM((_PB4, _L), jnp.float32),
                        pltpu.VMEM((_PB4, _L), jnp.float32),
                        pltpu.VMEM((8, _L), jnp.float32)],
    )(g3, a, b2, gam2, bet2)


def kernel(x, batch, W, b, gamma, beta):
    batch = batch.astype(jnp.int32)
    xt = x.T
    b_col = batch.reshape(_N, 1)
    b_row = batch.reshape(1, _N)

    bids = jnp.arange(_B, dtype=jnp.int32)
    starts = jnp.searchsorted(batch, bids, side="left").astype(jnp.int32)
    ends = jnp.searchsorted(batch, bids, side="right").astype(jnp.int32)
    bfirst = batch[::_RB]
    blast = batch[_RB - 1::_RB]
    jlo = (starts[bfirst] // _CT).astype(jnp.int32)
    jhi = ((ends[blast] - 1) // _CT).astype(jnp.int32)

    idx = _knn(x, xt, b_col, b_row, jlo, jhi)

    w2 = W[:, _IN:]
    wc = jnp.concatenate([(W[:, :_IN] - w2).T, w2.T], axis=1)
    a, bm = _ab_matmul(x, wc)

    g = _sc_gather(bm, idx.T.reshape(-1))
    g3 = g.reshape(_K, _N, _LH)

    return _bn_out(g3, a, b.reshape(1, _L), gamma.reshape(1, _L),
                   beta.reshape(1, _L))
